# baseline (speedup 1.0000x reference)
import functools

import jax
import jax.numpy as jnp
from jax import lax
from jax.experimental import pallas as pl
from jax.experimental.pallas import tpu as pltpu
from jax.experimental.pallas import tpu_sc as plsc

NC = 2
NS = 16
NW = NC * NS

LANES = 16
GRP = 16
PSTEP = 512
GPS = PSTEP // GRP


def _pair_table_body(e_ref, w_ref, b_ref, o_ref):
    h = (
        jnp.dot(e_ref[...], w_ref[...], preferred_element_type=jnp.float32)
        + b_ref[...]
    )
    v = h.shape[0]
    d = h.shape[1]
    a = jnp.broadcast_to(h[:, None, :], (v, v, d))
    bb = jnp.broadcast_to(h[None, :, :], (v, v, d))
    o_ref[...] = jnp.concatenate([a, bb], axis=-1)


def _make_sc_gather(n, v, d):
    d2 = 2 * d
    assert d2 == LANES
    npair = n // 2
    per_w = npair // NW
    nstep = per_w // PSTEP
    ntok_w = per_w * 2
    orows_step = PSTEP * d2 // 128
    orow_w = per_w * d2 // 128
    out_rows = npair * d2 // 128
    tsz = v * v * d2
    assert npair * 2 == n and per_w * NW == npair and nstep * PSTEP == per_w

    mesh = plsc.VectorSubcoreMesh(core_axis_name="c", subcore_axis_name="s")

    @functools.partial(
        pl.kernel,
        out_type=jax.ShapeDtypeStruct((out_rows, 128), jnp.float32),
        mesh=mesh,
        scratch_types=[
            pltpu.VMEM((ntok_w,), jnp.int32),
            pltpu.VMEM((tsz,), jnp.float32),
            pltpu.VMEM((orows_step, 128), jnp.float32),
        ],
        compiler_params=pltpu.CompilerParams(
            use_tc_tiling_on_sc=True, needs_layout_passes=False
        ),
    )
    def sc_gather(x_hbm, t2_hbm, out_hbm, xbig, t2t, rows):
        wid = lax.axis_index("s") * NC + lax.axis_index("c")
        pltpu.sync_copy(t2_hbm, t2t)
        pltpu.sync_copy(x_hbm.at[pl.ds(wid * ntok_w, ntok_w)], xbig)

        iota = lax.iota(jnp.int32, LANES)
        iota2 = iota * 2
        r1base = (iota & 7) * d2
        rhalf = iota >> 3
        obase = wid * orow_w

        def step(s, carry):
            @plsc.parallel_loop(0, GPS, unroll=2)
            def group(g):
                tbase = (s * PSTEP + g * GRP) * 2
                te = tbase + iota2
                ev = plsc.load_gather(xbig, [te])
                od = plsc.load_gather(xbig, [te + 1])
                p16 = (ev * v + od) * d2
                r0v = rhalf + g * 2
                for c in range(d2):
                    vals = plsc.load_gather(t2t, [p16 + c])
                    plsc.store_scatter(rows, [r0v, r1base + c], vals)

            pltpu.sync_copy(
                rows, out_hbm.at[pl.ds(obase + s * orows_step, orows_step)]
            )
            return carry

        lax.fori_loop(0, nstep, step, 0)

    return sc_gather


def kernel(x, embed_table, W, b):
    bs, sl = x.shape
    n = bs * sl
    v = embed_table.shape[0]
    d = W.shape[1]
    t2 = pl.pallas_call(
        _pair_table_body,
        out_shape=jax.ShapeDtypeStruct((v, v, 2 * d), jnp.float32),
    )(embed_table, W, b.reshape(1, d))
    xf = lax.optimization_barrier(x.reshape(n))
    out = _make_sc_gather(n, v, d)(xf, t2.reshape(v * v * 2 * d))
    return lax.optimization_barrier(out).reshape(bs, sl, d)


# speedup vs baseline: 1.1824x; 1.1824x over previous
import functools

import jax
import jax.numpy as jnp
from jax import lax
from jax.experimental import pallas as pl
from jax.experimental.pallas import tpu as pltpu
from jax.experimental.pallas import tpu_sc as plsc

NC = 2
NS = 16
NW = NC * NS

LANES = 16
BLK_I = 16
TC_BLK = 4


def _pair_table_body(e_ref, w_ref, b_ref, o_ref):
    h = (
        jnp.dot(e_ref[...], w_ref[...], preferred_element_type=jnp.float32)
        + b_ref[...]
    )
    v, d = h.shape
    a = jnp.broadcast_to(h[:, None, :], (v, v, d))
    bb = jnp.broadcast_to(h[None, :, :], (v, v, d))
    o_ref[...] = jnp.concatenate([a, bb], axis=-1)


def _make_sc_gather(bs, sl, v, d):
    d2 = 2 * d
    assert d2 == LANES and sl % 2 == 0
    n = bs * sl
    rows_i = bs // NW
    tok_w = rows_i * sl
    nblk = rows_i // BLK_I
    nhalf = 2
    blk_h = nblk // nhalf
    half_tok = tok_w // nhalf
    gps = sl // 2
    brows = BLK_I * sl * d // 128
    out_rows = n * d // 128
    tsz = v * v * d2
    assert brows * 128 == BLK_I * sl * d

    mesh = plsc.VectorSubcoreMesh(core_axis_name="c", subcore_axis_name="s")

    @functools.partial(
        pl.kernel,
        out_type=jax.ShapeDtypeStruct((out_rows, 128), jnp.float32),
        mesh=mesh,
        scratch_types=[
            pltpu.VMEM((half_tok,), jnp.int32),
            pltpu.VMEM((tsz,), jnp.float32),
            pltpu.VMEM((brows, 128), jnp.float32),
        ],
        compiler_params=pltpu.CompilerParams(
            use_tc_tiling_on_sc=True, needs_layout_passes=False
        ),
    )
    def sc_gather(x_hbm, t2_hbm, out_hbm, xbig, t2t, rows):
        wid = lax.axis_index("s") * NC + lax.axis_index("c")
        pltpu.sync_copy(t2_hbm, t2t)

        iota = lax.iota(jnp.int32, LANES)
        iota_sl = iota * sl
        i8 = iota * 8
        zero = iota * 0
        obase = wid * nblk * brows

        def half(h, carry):
            pltpu.sync_copy(
                x_hbm.at[pl.ds(wid * tok_w + h * half_tok, half_tok)], xbig
            )

            def block(bi, carry2):
                @plsc.parallel_loop(0, gps, unroll=2)
                def group(g):
                    te = bi * (BLK_I * sl) + 2 * g + iota_sl
                    ev = plsc.load_gather(xbig, [te])
                    od = plsc.load_gather(xbig, [te + 1])
                    p16 = (ev * v + od) * d2
                    for c in range(d2):
                        vals = plsc.load_gather(t2t, [p16 + c])
                        r = zero + (2 * g + (1 if c >= d else 0))
                        plsc.store_scatter(rows, [r, i8 + (c % d)], vals)

                pltpu.sync_copy(
                    rows,
                    out_hbm.at[
                        pl.ds(obase + (h * blk_h + bi) * brows, brows)
                    ],
                )
                return carry2

            lax.fori_loop(0, blk_h, block, 0)
            return carry

        lax.fori_loop(0, nhalf, half, 0)

    return sc_gather


def _relayout_body(i_ref, o_ref):
    x = i_ref[...]
    nb, sl, d = o_ref.shape
    for bi in range(nb // BLK_I):
        for l in range(BLK_I):
            o_ref[bi * BLK_I + l] = x[bi * sl : (bi + 1) * sl, l * d : (l + 1) * d]


def _relayout(lin, bs, sl, d):
    rows_blk = TC_BLK * BLK_I * sl * d // 128
    grid = bs // (TC_BLK * BLK_I)
    return pl.pallas_call(
        _relayout_body,
        grid=(grid,),
        in_specs=[pl.BlockSpec((rows_blk, 128), lambda g: (g, 0))],
        out_specs=pl.BlockSpec((TC_BLK * BLK_I, sl, d), lambda g: (g, 0, 0)),
        out_shape=jax.ShapeDtypeStruct((bs, sl, d), jnp.float32),
    )(lin)


def kernel(x, embed_table, W, b):
    bs, sl = x.shape
    v = embed_table.shape[0]
    d = W.shape[1]
    t2 = pl.pallas_call(
        _pair_table_body,
        out_shape=jax.ShapeDtypeStruct((v, v, 2 * d), jnp.float32),
    )(embed_table, W, b.reshape(1, d))
    xf = lax.optimization_barrier(x.reshape(bs * sl))
    lin = _make_sc_gather(bs, sl, v, d)(xf, t2.reshape(v * v * 2 * d))
    return _relayout(lin, bs, sl, d)


# speedup vs baseline: 7.8952x; 6.6773x over previous
import functools

import jax
import jax.numpy as jnp
from jax import lax
from jax.experimental import pallas as pl
from jax.experimental.pallas import tpu as pltpu
from jax.experimental.pallas import tpu_sc as plsc

NC = 2
NS = 16
NW = NC * NS

LANES = 16
ISLAB = 128
JQ = 50


def _pair_table_body(e_ref, w_ref, b_ref, o_ref):
    h = (
        jnp.dot(e_ref[...], w_ref[...], preferred_element_type=jnp.float32)
        + b_ref[...]
    )
    v, d = h.shape
    a = jnp.broadcast_to(h[:, None, :], (v, v, d))
    bb = jnp.broadcast_to(h[None, :, :], (v, v, d))
    o_ref[...] = jnp.concatenate([a, bb], axis=-1)


def _make_sc_gather(bs, sl, v, d):
    d2 = 2 * d
    assert d2 == LANES and sl % (2 * JQ) == 0 and bs % (NW * ISLAB) == 0
    rows_i = bs // NW
    nslab_i = rows_i // ISLAB
    nslab_j = sl // JQ
    slab_tok = ISLAB * sl
    gp = JQ // 2
    nsub = ISLAB // LANES
    ngrp = gp * nsub
    tsz = v * v * d2

    mesh = plsc.VectorSubcoreMesh(core_axis_name="c", subcore_axis_name="s")

    @functools.partial(
        pl.kernel,
        out_type=jax.ShapeDtypeStruct((sl, d, bs), jnp.float32),
        mesh=mesh,
        scratch_types=[
            pltpu.VMEM((slab_tok,), jnp.int32),
            pltpu.VMEM((tsz,), jnp.float32),
            pltpu.VMEM((JQ, d, ISLAB), jnp.float32),
        ],
        compiler_params=pltpu.CompilerParams(
            use_tc_tiling_on_sc=True, needs_layout_passes=False
        ),
    )
    def sc_gather(x_hbm, t2_hbm, out_hbm, xbig, t2t, rows):
        wid = lax.axis_index("s") * NC + lax.axis_index("c")
        pltpu.sync_copy(t2_hbm, t2t)

        iota = lax.iota(jnp.int32, LANES)
        iota_sl = iota * sl
        zero = iota * 0
        wi0 = wid * rows_i

        def islab(si, carry):
            pltpu.sync_copy(
                x_hbm.at[pl.ds((wi0 + si * ISLAB) * sl, slab_tok)], xbig
            )

            def jslab(jq, carry2):
                @plsc.parallel_loop(0, ngrp, unroll=2)
                def group(q):
                    g = q >> 3
                    sub = q & (nsub - 1)
                    te = sub * (LANES * sl) + jq * JQ + 2 * g + iota_sl
                    ev = plsc.load_gather(xbig, [te])
                    od = plsc.load_gather(xbig, [te + 1])
                    p16 = (ev * v + od) * d2
                    ivec = sub * LANES + iota
                    for c in range(d2):
                        vals = plsc.load_gather(t2t, [p16 + c])
                        jv = zero + (2 * g + (1 if c >= d else 0))
                        kv = zero + (c % d)
                        plsc.store_scatter(rows, [jv, kv, ivec], vals)

                pltpu.sync_copy(
                    rows,
                    out_hbm.at[
                        pl.ds(jq * JQ, JQ),
                        slice(None),
                        pl.ds(wi0 + si * ISLAB, ISLAB),
                    ],
                )
                return carry2

            lax.fori_loop(0, nslab_j, jslab, 0)
            return carry

        lax.fori_loop(0, nslab_i, islab, 0)

    return sc_gather


def kernel(x, embed_table, W, b):
    bs, sl = x.shape
    v = embed_table.shape[0]
    d = W.shape[1]
    t2 = pl.pallas_call(
        _pair_table_body,
        out_shape=jax.ShapeDtypeStruct((v, v, 2 * d), jnp.float32),
    )(embed_table, W, b.reshape(1, d))
    xf = lax.optimization_barrier(x.reshape(bs * sl))
    jki = _make_sc_gather(bs, sl, v, d)(xf, t2.reshape(v * v * 2 * d))
    return jnp.transpose(jki, (2, 0, 1))


# speedup vs baseline: 8.3854x; 1.0621x over previous
import functools

import jax
import jax.numpy as jnp
from jax import lax
from jax.experimental import pallas as pl
from jax.experimental.pallas import tpu as pltpu
from jax.experimental.pallas import tpu_sc as plsc

NC = 2
NS = 16
NW = NC * NS

LANES = 16
ISLAB = 128
JQ = 50


def _pair_table_body(e_ref, w_ref, b_ref, o_ref):
    h = (
        jnp.dot(e_ref[...], w_ref[...], preferred_element_type=jnp.float32)
        + b_ref[...]
    )
    v, d = h.shape
    a = jnp.broadcast_to(h[:, None, :], (v, v, d))
    bb = jnp.broadcast_to(h[None, :, :], (v, v, d))
    o_ref[...] = jnp.concatenate([a, bb], axis=-1)


def _make_sc_gather(bs, sl, v, d):
    d2 = 2 * d
    assert d2 == LANES and sl % (2 * JQ) == 0 and bs % (NW * ISLAB) == 0
    rows_i = bs // NW
    nslab_i = rows_i // ISLAB
    nslab_j = sl // JQ
    slab_tok = ISLAB * sl
    gp = JQ // 2
    nsub = ISLAB // LANES
    ngrp = gp * nsub
    tsz = v * v * d2

    mesh = plsc.VectorSubcoreMesh(core_axis_name="c", subcore_axis_name="s")

    @functools.partial(
        pl.kernel,
        out_type=jax.ShapeDtypeStruct((sl, d, bs), jnp.float32),
        mesh=mesh,
        scratch_types=[
            pltpu.VMEM((sl, ISLAB), jnp.int32),
            pltpu.VMEM((tsz,), jnp.float32),
            pltpu.VMEM((JQ, d, ISLAB), jnp.float32),
        ],
        compiler_params=pltpu.CompilerParams(
            use_tc_tiling_on_sc=True, needs_layout_passes=False
        ),
    )
    def sc_gather(x_hbm, t2_hbm, out_hbm, xbig, t2t, rows):
        wid = lax.axis_index("s") * NC + lax.axis_index("c")
        pltpu.sync_copy(t2_hbm, t2t)

        iota = lax.iota(jnp.int32, LANES)
        zero = iota * 0
        kvs = [zero + k for k in range(d)]
        wi0 = wid * rows_i

        def islab(si, carry):
            pltpu.sync_copy(
                x_hbm.at[slice(None), pl.ds(wi0 + si * ISLAB, ISLAB)], xbig
            )

            def jslab(jq, carry2):
                @plsc.parallel_loop(0, ngrp, unroll=2)
                def group(q):
                    g = q >> 3
                    sub = q & (nsub - 1)
                    ivec = sub * LANES + iota
                    jv0 = zero + (jq * JQ + 2 * g)
                    jv1 = jv0 + 1
                    ev = plsc.load_gather(xbig, [jv0, ivec])
                    od = plsc.load_gather(xbig, [jv1, ivec])
                    p16 = (ev * v + od) * d2
                    rj0 = zero + 2 * g
                    rj1 = rj0 + 1
                    for c in range(d2):
                        vals = plsc.load_gather(t2t, [p16 + c])
                        plsc.store_scatter(
                            rows,
                            [rj0 if c < d else rj1, kvs[c % d], ivec],
                            vals,
                        )

                pltpu.sync_copy(
                    rows,
                    out_hbm.at[
                        pl.ds(jq * JQ, JQ),
                        slice(None),
                        pl.ds(wi0 + si * ISLAB, ISLAB),
                    ],
                )
                return carry2

            lax.fori_loop(0, nslab_j, jslab, 0)
            return carry

        lax.fori_loop(0, nslab_i, islab, 0)

    return sc_gather


def kernel(x, embed_table, W, b):
    bs, sl = x.shape
    v = embed_table.shape[0]
    d = W.shape[1]
    t2 = pl.pallas_call(
        _pair_table_body,
        out_shape=jax.ShapeDtypeStruct((v, v, 2 * d), jnp.float32),
    )(embed_table, W, b.reshape(1, d))
    xt = jnp.transpose(x)
    jki = _make_sc_gather(bs, sl, v, d)(xt, t2.reshape(v * v * 2 * d))
    return jnp.transpose(jki, (2, 0, 1))
